# baseline (speedup 1.0000x reference)
import jax
import jax.numpy as jnp
from jax import lax
from jax.experimental import pallas as pl
from jax.experimental.pallas import tpu as pltpu

IN_DIM = 50000
H = 64
SEQ_LEN = 10
E = 800000
ES = E // SEQ_LEN

RBN = 6272
NRB = 8
CN = 6400
NCB = 8
NEG = -1e30


def _pool_body(agg_ref, deg_ref, ent_ref, wself_ref, out_ref):
    rb = pl.program_id(0)
    s = pl.program_id(1)

    xs = jnp.dot(ent_ref[...], wself_ref[...],
                 preferred_element_type=jnp.float32)
    a = agg_ref[0]
    d = deg_ref[0]
    hn = jnp.maximum(a / jnp.maximum(d, 1.0) + xs, 0.0)
    rows = rb * RBN + lax.broadcasted_iota(jnp.int32, (RBN, 1), 0)
    hn = jnp.where(rows < IN_DIM, hn, NEG)
    bm = jnp.max(hn, axis=0, keepdims=True)

    @pl.when(rb == 0)
    def _():
        out_ref[pl.ds(s, 1), :] = bm

    @pl.when(rb != 0)
    def _():
        out_ref[pl.ds(s, 1), :] = jnp.maximum(out_ref[pl.ds(s, 1), :], bm)


def _pool(agg, deg3, ent, W_self):
    return pl.pallas_call(
        _pool_body,
        grid=(NRB, SEQ_LEN),
        in_specs=[
            pl.BlockSpec((1, RBN, H), lambda rb, s: (s, rb, 0)),
            pl.BlockSpec((1, RBN, 1), lambda rb, s: (s, rb, 0)),
            pl.BlockSpec((RBN, H), lambda rb, s: (rb, 0)),
            pl.BlockSpec((H, H), lambda rb, s: (0, 0)),
        ],
        out_specs=pl.BlockSpec((SEQ_LEN, H), lambda rb, s: (0, 0)),
        out_shape=jax.ShapeDtypeStruct((SEQ_LEN, H), jnp.float32),
    )(agg, deg3, ent, W_self)


def _head_body(seq_ref, wih_ref, whh_ref, bih_ref, bhh_ref,
               wlin_ref, blin_ref, tpo_ref, out_ref,
               hvec, macc, sacc, dacc):
    cb = pl.program_id(0)

    @pl.when(cb == 0)
    def _():
        hst = jnp.zeros((1, H), jnp.float32)
        for i in range(SEQ_LEN):
            x = seq_ref[pl.ds(i, 1), :]
            gi = jnp.dot(x, wih_ref[...],
                         preferred_element_type=jnp.float32) + bih_ref[...]
            gh = jnp.dot(hst, whh_ref[...],
                         preferred_element_type=jnp.float32) + bhh_ref[...]
            r = jax.nn.sigmoid(gi[:, :H] + gh[:, :H])
            z = jax.nn.sigmoid(gi[:, H:2 * H] + gh[:, H:2 * H])
            n = jnp.tanh(gi[:, 2 * H:] + r * gh[:, 2 * H:])
            hst = (1.0 - z) * n + z * hst
        hvec[...] = hst
        macc[...] = jnp.full((1, 128), NEG, jnp.float32)
        sacc[...] = jnp.zeros((1, 128), jnp.float32)
        dacc[...] = jnp.zeros((1, 128), jnp.float32)

    h = hvec[...]
    pred = jnp.dot(h, wlin_ref[...],
                   preferred_element_type=jnp.float32) + blin_ref[...]
    cols = cb * CN + lax.broadcasted_iota(jnp.int32, (1, CN), 1)
    valid = cols < IN_DIM
    p = jnp.where(valid, pred, NEG).reshape(CN // 128, 128)

    mb = jnp.max(p, axis=0, keepdims=True)
    mnew = jnp.maximum(macc[...], mb)
    sacc[...] = (sacc[...] * jnp.exp(macc[...] - mnew)
                 + jnp.sum(jnp.exp(p - mnew), axis=0, keepdims=True))
    macc[...] = mnew

    tb = jnp.mean(tpo_ref[...], axis=0, keepdims=True)
    contrib = jnp.where(valid, tb * pred, 0.0).reshape(CN // 128, 128)
    dacc[...] = dacc[...] + jnp.sum(contrib, axis=0, keepdims=True)

    @pl.when(cb == NCB - 1)
    def _():
        m = jnp.max(macc[...])
        ssum = jnp.sum(sacc[...] * jnp.exp(macc[...] - m))
        lse = m + jnp.log(ssum)
        out_ref[...] = jnp.reshape(lse - jnp.sum(dacc[...]), (1, 1))


def _head(seq, W_ih, W_hh, b_ih, b_hh, W_lin, b_lin, tpo):
    return pl.pallas_call(
        _head_body,
        grid=(NCB,),
        in_specs=[
            pl.BlockSpec((SEQ_LEN, H), lambda cb: (0, 0)),
            pl.BlockSpec((H, 3 * H), lambda cb: (0, 0)),
            pl.BlockSpec((H, 3 * H), lambda cb: (0, 0)),
            pl.BlockSpec((1, 3 * H), lambda cb: (0, 0)),
            pl.BlockSpec((1, 3 * H), lambda cb: (0, 0)),
            pl.BlockSpec((H, CN), lambda cb: (0, cb)),
            pl.BlockSpec((1, CN), lambda cb: (0, cb)),
            pl.BlockSpec((64, CN), lambda cb: (0, cb)),
        ],
        out_specs=pl.BlockSpec((1, 1), lambda cb: (0, 0)),
        out_shape=jax.ShapeDtypeStruct((1, 1), jnp.float32),
        scratch_shapes=[
            pltpu.VMEM((1, H), jnp.float32),
            pltpu.VMEM((1, 128), jnp.float32),
            pltpu.VMEM((1, 128), jnp.float32),
            pltpu.VMEM((1, 128), jnp.float32),
        ],
    )(seq, W_ih, W_hh, b_ih, b_hh, W_lin, b_lin, tpo)


def kernel(t_list, true_prob_s, true_prob_o, edge_index, edge_type, ent_embeds,
           w_rel, W_self, W_ih, W_hh, b_ih, b_hh, W_lin, b_lin):
    src = edge_index[0].astype(jnp.int32).reshape(SEQ_LEN, ES)
    dst = edge_index[1].astype(jnp.int32).reshape(SEQ_LEN, ES)
    rt = edge_type.astype(jnp.int32).reshape(SEQ_LEN, ES)

    msg = jnp.take(ent_embeds, src, axis=0) * jnp.take(w_rel, rt, axis=0)
    snap = jnp.arange(SEQ_LEN, dtype=jnp.int32)[:, None]
    agg = jnp.zeros((SEQ_LEN, IN_DIM, H), jnp.float32).at[snap, dst].add(msg)
    deg = jnp.zeros((SEQ_LEN, IN_DIM), jnp.float32).at[snap, dst].add(1.0)

    seq = _pool(agg, deg[..., None], ent_embeds, W_self)
    loss = _head(seq, W_ih, W_hh, b_ih[None, :], b_hh[None, :],
                 W_lin, b_lin[None, :], true_prob_o)
    return loss[0, 0]


# speedup vs baseline: 1.6908x; 1.6908x over previous
import jax
import jax.numpy as jnp
from jax import lax
from jax.experimental import pallas as pl
from jax.experimental.pallas import tpu as pltpu

IN_DIM = 50000
H = 64
SEQ_LEN = 10
E = 800000
ES = E // SEQ_LEN

RBN = 6272
NRB = 8
CN = 6400
NCB = 8
NEG = -1e30


def _pool_body(agg_ref, deg_ref, ent_ref, wself_ref, out_ref):
    rb = pl.program_id(0)
    s = pl.program_id(1)

    xs = jnp.dot(ent_ref[...], wself_ref[...],
                 preferred_element_type=jnp.float32)
    a = agg_ref[0]
    d = deg_ref[0]
    hn = jnp.maximum(a / jnp.maximum(d, 1.0) + xs, 0.0)
    rows = rb * RBN + lax.broadcasted_iota(jnp.int32, (RBN, 1), 0)
    hn = jnp.where(rows < IN_DIM, hn, NEG)
    bm = jnp.max(hn, axis=0, keepdims=True)

    @pl.when(rb == 0)
    def _():
        out_ref[pl.ds(s, 1), :] = bm

    @pl.when(rb != 0)
    def _():
        out_ref[pl.ds(s, 1), :] = jnp.maximum(out_ref[pl.ds(s, 1), :], bm)


def _pool(agg, deg3, ent, W_self):
    return pl.pallas_call(
        _pool_body,
        grid=(NRB, SEQ_LEN),
        in_specs=[
            pl.BlockSpec((1, RBN, H), lambda rb, s: (s, rb, 0)),
            pl.BlockSpec((1, RBN, 1), lambda rb, s: (s, rb, 0)),
            pl.BlockSpec((RBN, H), lambda rb, s: (rb, 0)),
            pl.BlockSpec((H, H), lambda rb, s: (0, 0)),
        ],
        out_specs=pl.BlockSpec((SEQ_LEN, H), lambda rb, s: (0, 0)),
        out_shape=jax.ShapeDtypeStruct((SEQ_LEN, H), jnp.float32),
    )(agg, deg3, ent, W_self)


def _head_body(seq_ref, wih_ref, whh_ref, bih_ref, bhh_ref,
               wlin_ref, blin_ref, tpo_ref, out_ref,
               hvec, macc, sacc, dacc):
    cb = pl.program_id(0)

    @pl.when(cb == 0)
    def _():
        hst = jnp.zeros((1, H), jnp.float32)
        for i in range(SEQ_LEN):
            x = seq_ref[pl.ds(i, 1), :]
            gi = jnp.dot(x, wih_ref[...],
                         preferred_element_type=jnp.float32) + bih_ref[...]
            gh = jnp.dot(hst, whh_ref[...],
                         preferred_element_type=jnp.float32) + bhh_ref[...]
            r = jax.nn.sigmoid(gi[:, :H] + gh[:, :H])
            z = jax.nn.sigmoid(gi[:, H:2 * H] + gh[:, H:2 * H])
            n = jnp.tanh(gi[:, 2 * H:] + r * gh[:, 2 * H:])
            hst = (1.0 - z) * n + z * hst
        hvec[...] = hst
        macc[...] = jnp.full((1, 128), NEG, jnp.float32)
        sacc[...] = jnp.zeros((1, 128), jnp.float32)
        dacc[...] = jnp.zeros((1, 128), jnp.float32)

    h = hvec[...]
    pred = jnp.dot(h, wlin_ref[...],
                   preferred_element_type=jnp.float32) + blin_ref[...]
    cols = cb * CN + lax.broadcasted_iota(jnp.int32, (1, CN), 1)
    valid = cols < IN_DIM
    p = jnp.where(valid, pred, NEG).reshape(CN // 128, 128)

    mb = jnp.max(p, axis=0, keepdims=True)
    mnew = jnp.maximum(macc[...], mb)
    sacc[...] = (sacc[...] * jnp.exp(macc[...] - mnew)
                 + jnp.sum(jnp.exp(p - mnew), axis=0, keepdims=True))
    macc[...] = mnew

    tb = jnp.mean(tpo_ref[...], axis=0, keepdims=True)
    contrib = jnp.where(valid, tb * pred, 0.0).reshape(CN // 128, 128)
    dacc[...] = dacc[...] + jnp.sum(contrib, axis=0, keepdims=True)

    @pl.when(cb == NCB - 1)
    def _():
        m = jnp.max(macc[...])
        ssum = jnp.sum(sacc[...] * jnp.exp(macc[...] - m))
        lse = m + jnp.log(ssum)
        out_ref[...] = jnp.reshape(lse - jnp.sum(dacc[...]), (1, 1))


def _head(seq, W_ih, W_hh, b_ih, b_hh, W_lin, b_lin, tpo):
    return pl.pallas_call(
        _head_body,
        grid=(NCB,),
        in_specs=[
            pl.BlockSpec((SEQ_LEN, H), lambda cb: (0, 0)),
            pl.BlockSpec((H, 3 * H), lambda cb: (0, 0)),
            pl.BlockSpec((H, 3 * H), lambda cb: (0, 0)),
            pl.BlockSpec((1, 3 * H), lambda cb: (0, 0)),
            pl.BlockSpec((1, 3 * H), lambda cb: (0, 0)),
            pl.BlockSpec((H, CN), lambda cb: (0, cb)),
            pl.BlockSpec((1, CN), lambda cb: (0, cb)),
            pl.BlockSpec((64, CN), lambda cb: (0, cb)),
        ],
        out_specs=pl.BlockSpec((1, 1), lambda cb: (0, 0)),
        out_shape=jax.ShapeDtypeStruct((1, 1), jnp.float32),
        scratch_shapes=[
            pltpu.VMEM((1, H), jnp.float32),
            pltpu.VMEM((1, 128), jnp.float32),
            pltpu.VMEM((1, 128), jnp.float32),
            pltpu.VMEM((1, 128), jnp.float32),
        ],
    )(seq, W_ih, W_hh, b_ih, b_hh, W_lin, b_lin, tpo)


def kernel(t_list, true_prob_s, true_prob_o, edge_index, edge_type, ent_embeds,
           w_rel, W_self, W_ih, W_hh, b_ih, b_hh, W_lin, b_lin):
    src = edge_index[0].astype(jnp.int32).reshape(SEQ_LEN, ES)
    dst = edge_index[1].astype(jnp.int32).reshape(SEQ_LEN, ES)
    rt = edge_type.astype(jnp.int32).reshape(SEQ_LEN, ES)

    aggs = []
    degs = []
    for s in range(SEQ_LEN):
        msg = (jnp.take(ent_embeds, src[s], axis=0)
               * jnp.take(w_rel, rt[s], axis=0))
        aggs.append(jnp.zeros((IN_DIM, H), jnp.float32).at[dst[s]].add(msg))
        degs.append(jnp.zeros((IN_DIM,), jnp.float32).at[dst[s]].add(1.0))
    agg = jnp.stack(aggs, axis=0)
    deg = jnp.stack(degs, axis=0)

    seq = _pool(agg, deg[..., None], ent_embeds, W_self)
    loss = _head(seq, W_ih, W_hh, b_ih[None, :], b_hh[None, :],
                 W_lin, b_lin[None, :], true_prob_o)
    return loss[0, 0]
